# Initial kernel scaffold; baseline (speedup 1.0000x reference)
import jax
import jax.numpy as jnp
from jax.experimental import pallas as pl


def kernel(x, y):
    raise NotImplementedError("write your pallas kernel here")


# speedup vs baseline: 1.0533x; 1.0533x over previous
import functools

import jax
import jax.numpy as jnp
from jax import lax
from jax.experimental import pallas as pl
from jax.experimental.pallas import tpu as pltpu
from jax.experimental.pallas import tpu_sc as plsc

B = 128
N = 32768
NC = 2
NS = 16
L = 16
NW = NC * NS
ROWS_PER_W = B // NW
U = 8

_mesh = plsc.VectorSubcoreMesh(core_axis_name="c", subcore_axis_name="s")


@functools.partial(
    pl.kernel,
    mesh=_mesh,
    out_type=jax.ShapeDtypeStruct((B, L), jnp.float32),
    scratch_types=[
        pltpu.VMEM((N,), jnp.float32),
        pltpu.VMEM((N,), jnp.float32),
        pltpu.VMEM((ROWS_PER_W, L), jnp.float32),
        pltpu.SemaphoreType.DMA,
        pltpu.SemaphoreType.DMA,
    ],
)
def _hinge_sc(x_hbm, y_hbm, out_hbm, xb, yb, ob, sem_x, sem_y):
    cid = lax.axis_index("c")
    sid = lax.axis_index("s")
    wid = sid * NC + cid
    row0 = wid * ROWS_PER_W

    def chunk_body(i, ms):
        base = i * (L * U)
        out = []
        for u in range(U):
            xv = xb[pl.ds(base + u * L, L)]
            yv = yb[pl.ds(base + u * L, L)]
            out.append(jnp.maximum(ms[u], jnp.where(yv >= 0.5, xv, 0.0)))
        return tuple(out)

    for r in range(ROWS_PER_W):
        cx = pltpu.async_copy(x_hbm.at[row0 + r], xb, sem_x)
        cy = pltpu.async_copy(y_hbm.at[row0 + r], yb, sem_y)
        cx.wait()
        cy.wait()
        init = tuple(jnp.full((L,), -jnp.inf, jnp.float32) for _ in range(U))
        ms = lax.fori_loop(0, N // (L * U), chunk_body, init)
        m = ms[0]
        for u in range(1, U):
            m = jnp.maximum(m, ms[u])
        ob[r, :] = m

    pltpu.sync_copy(ob, out_hbm.at[pl.ds(row0, ROWS_PER_W)])


def _sum_tc(p_ref, o_ref):
    o_ref[0, 0] = jnp.sum(jnp.max(p_ref[...], axis=1)) * (1.0 / B)


_reduce = pl.pallas_call(
    _sum_tc,
    out_shape=jax.ShapeDtypeStruct((1, 1), jnp.float32),
    out_specs=pl.BlockSpec(memory_space=pltpu.SMEM),
)


@jax.jit
def kernel(x, y):
    partials = _hinge_sc(x, y)
    return _reduce(partials)[0, 0]


# speedup vs baseline: 1.2617x; 1.1979x over previous
import functools

import jax
import jax.numpy as jnp
from jax import lax
from jax.experimental import pallas as pl
from jax.experimental.pallas import tpu as pltpu
from jax.experimental.pallas import tpu_sc as plsc

B = 128
N = 32768
NC = 2
NS = 16
L = 16
NW = NC * NS
ROWS_PER_W = B // NW
U = 8
C = 16384
CPR = N // C
T = ROWS_PER_W * CPR

_mesh = plsc.VectorSubcoreMesh(core_axis_name="c", subcore_axis_name="s")


@functools.partial(
    pl.kernel,
    mesh=_mesh,
    out_type=jax.ShapeDtypeStruct((B, L), jnp.float32),
    scratch_types=[
        pltpu.VMEM((C,), jnp.float32),
        pltpu.VMEM((C,), jnp.float32),
        pltpu.VMEM((C,), jnp.float32),
        pltpu.VMEM((C,), jnp.float32),
        pltpu.VMEM((ROWS_PER_W, L), jnp.float32),
        pltpu.SemaphoreType.DMA,
        pltpu.SemaphoreType.DMA,
        pltpu.SemaphoreType.DMA,
        pltpu.SemaphoreType.DMA,
    ],
)
def _hinge_sc(x_hbm, y_hbm, out_hbm, xb0, xb1, yb0, yb1, ob,
              sx0, sx1, sy0, sy1):
    cid = lax.axis_index("c")
    sid = lax.axis_index("s")
    wid = sid * NC + cid
    row0 = wid * ROWS_PER_W

    xbufs, ybufs = (xb0, xb1), (yb0, yb1)
    sxs, sys_ = (sx0, sx1), (sy0, sy1)
    pend = [None, None]

    def start(t):
        s = t % 2
        r = row0 + t // CPR
        off = (t % CPR) * C
        pend[s] = (
            pltpu.async_copy(x_hbm.at[r, pl.ds(off, C)], xbufs[s], sxs[s]),
            pltpu.async_copy(y_hbm.at[r, pl.ds(off, C)], ybufs[s], sys_[s]),
        )

    def make_chunk(xb, yb):
        def body(i, ms):
            out = []
            for u in range(U):
                xv = xb[pl.ds(i + u * L, L)]
                yv = yb[pl.ds(i + u * L, L)]
                out.append(jnp.maximum(ms[u], jnp.where(yv >= 0.5, xv, 0.0)))
            return tuple(out)
        return body

    start(0)
    ms = None
    for t in range(T):
        if t + 1 < T:
            start(t + 1)
        s = t % 2
        cx, cy = pend[s]
        cx.wait()
        cy.wait()
        if t % CPR == 0:
            ms = tuple(jnp.full((L,), -jnp.inf, jnp.float32) for _ in range(U))
        ms = plsc.parallel_loop(0, C, L * U, unroll=2, carry=ms)(
            make_chunk(xbufs[s], ybufs[s]))
        if t % CPR == CPR - 1:
            m = ms[0]
            for u in range(1, U):
                m = jnp.maximum(m, ms[u])
            ob[t // CPR, :] = m

    pltpu.sync_copy(ob, out_hbm.at[pl.ds(row0, ROWS_PER_W)])


def _sum_tc(p_ref, o_ref):
    o_ref[0, 0] = jnp.sum(jnp.max(p_ref[...], axis=1)) * (1.0 / B)


_reduce = pl.pallas_call(
    _sum_tc,
    out_shape=jax.ShapeDtypeStruct((1, 1), jnp.float32),
    out_specs=pl.BlockSpec(memory_space=pltpu.SMEM),
)


@jax.jit
def kernel(x, y):
    partials = _hinge_sc(x, y)
    return _reduce(partials)[0, 0]


# speedup vs baseline: 1.2762x; 1.0114x over previous
import functools

import jax
import jax.numpy as jnp
from jax import lax
from jax.experimental import pallas as pl
from jax.experimental.pallas import tpu as pltpu
from jax.experimental.pallas import tpu_sc as plsc

B = 128
N = 32768
NC = 2
NS = 16
L = 16
NW = NC * NS
ROWS_PER_W = B // NW
U = 8
C = 16384
CPR = N // C
T = ROWS_PER_W * CPR

_mesh = plsc.VectorSubcoreMesh(core_axis_name="c", subcore_axis_name="s")


@functools.partial(
    pl.kernel,
    mesh=_mesh,
    out_type=jax.ShapeDtypeStruct((B, L), jnp.float32),
    scratch_types=[
        pltpu.VMEM((C,), jnp.float32),
        pltpu.VMEM((C,), jnp.float32),
        pltpu.VMEM((C,), jnp.float32),
        pltpu.VMEM((C,), jnp.float32),
        pltpu.VMEM((ROWS_PER_W, L), jnp.float32),
        pltpu.SemaphoreType.DMA,
        pltpu.SemaphoreType.DMA,
        pltpu.SemaphoreType.DMA,
        pltpu.SemaphoreType.DMA,
    ],
)
def _hinge_sc(x_hbm, y_hbm, out_hbm, xb0, xb1, yb0, yb1, ob,
              sx0, sx1, sy0, sy1):
    cid = lax.axis_index("c")
    sid = lax.axis_index("s")
    wid = sid * NC + cid
    row0 = wid * ROWS_PER_W

    def start(r, off, xb, yb, sx, sy):
        pltpu.async_copy(x_hbm.at[r, pl.ds(off, C)], xb, sx)
        pltpu.async_copy(y_hbm.at[r, pl.ds(off, C)], yb, sy)

    def wait(xb, yb, sx, sy):
        pltpu.make_async_copy(x_hbm.at[0, pl.ds(0, C)], xb, sx).wait()
        pltpu.make_async_copy(y_hbm.at[0, pl.ds(0, C)], yb, sy).wait()

    def make_chunk(xb, yb):
        def body(i, ms):
            out = []
            for u in range(U):
                xv = xb[pl.ds(i + u * L, L)]
                yv = yb[pl.ds(i + u * L, L)]
                out.append(jnp.maximum(ms[u], jnp.where(yv >= 0.5, xv, 0.0)))
            return tuple(out)
        return body

    start(row0, 0, xb0, yb0, sx0, sy0)
    start(row0, C, xb1, yb1, sx1, sy1)

    def row_body(k, _):
        r = row0 + k
        init = tuple(jnp.full((L,), -jnp.inf, jnp.float32) for _ in range(U))
        wait(xb0, yb0, sx0, sy0)
        ms = plsc.parallel_loop(0, C, L * U, unroll=4, carry=init)(
            make_chunk(xb0, yb0))
        @pl.when(k < ROWS_PER_W - 1)
        def _():
            start(r + 1, 0, xb0, yb0, sx0, sy0)
        wait(xb1, yb1, sx1, sy1)
        ms = plsc.parallel_loop(0, C, L * U, unroll=4, carry=ms)(
            make_chunk(xb1, yb1))
        @pl.when(k < ROWS_PER_W - 1)
        def _():
            start(r + 1, C, xb1, yb1, sx1, sy1)
        m = ms[0]
        for u in range(1, U):
            m = jnp.maximum(m, ms[u])
        ob[k, :] = m
        return 0

    lax.fori_loop(0, ROWS_PER_W, row_body, 0)
    pltpu.sync_copy(ob, out_hbm.at[pl.ds(row0, ROWS_PER_W)])


def _sum_tc(p_ref, o_ref):
    o_ref[0, 0] = jnp.sum(jnp.max(p_ref[...], axis=1)) * (1.0 / B)


_reduce = pl.pallas_call(
    _sum_tc,
    out_shape=jax.ShapeDtypeStruct((1, 1), jnp.float32),
    out_specs=pl.BlockSpec(memory_space=pltpu.SMEM),
)


@jax.jit
def kernel(x, y):
    partials = _hinge_sc(x, y)
    return _reduce(partials)[0, 0]


# speedup vs baseline: 1.4395x; 1.1280x over previous
import functools

import jax
import jax.numpy as jnp
from jax import lax
from jax.experimental import pallas as pl
from jax.experimental.pallas import tpu as pltpu
from jax.experimental.pallas import tpu_sc as plsc

B = 128
N = 32768
NC = 2
NS = 16
L = 16
NW = NC * NS
B_SC = 64
B_TC = B - B_SC
RPW = B_SC // NW
U = 8
C = 16384
RB = 8

_mesh = plsc.VectorSubcoreMesh(core_axis_name="c", subcore_axis_name="s")


@functools.partial(
    pl.kernel,
    mesh=_mesh,
    out_type=jax.ShapeDtypeStruct((B_SC, L), jnp.float32),
    scratch_types=[
        pltpu.VMEM((C,), jnp.float32),
        pltpu.VMEM((C,), jnp.float32),
        pltpu.VMEM((C,), jnp.float32),
        pltpu.VMEM((C,), jnp.float32),
        pltpu.VMEM((RPW, L), jnp.float32),
        pltpu.SemaphoreType.DMA,
        pltpu.SemaphoreType.DMA,
        pltpu.SemaphoreType.DMA,
        pltpu.SemaphoreType.DMA,
    ],
)
def _hinge_sc(x_hbm, y_hbm, out_hbm, xb0, xb1, yb0, yb1, ob,
              sx0, sx1, sy0, sy1):
    cid = lax.axis_index("c")
    sid = lax.axis_index("s")
    wid = sid * NC + cid
    row0 = wid * RPW

    def start(r, off, xb, yb, sx, sy):
        pltpu.async_copy(x_hbm.at[r, pl.ds(off, C)], xb, sx)
        pltpu.async_copy(y_hbm.at[r, pl.ds(off, C)], yb, sy)

    def wait(xb, yb, sx, sy):
        pltpu.make_async_copy(x_hbm.at[0, pl.ds(0, C)], xb, sx).wait()
        pltpu.make_async_copy(y_hbm.at[0, pl.ds(0, C)], yb, sy).wait()

    def make_chunk(xb, yb):
        def body(i, ms):
            out = []
            for u in range(U):
                xv = xb[pl.ds(i + u * L, L)]
                yv = yb[pl.ds(i + u * L, L)]
                out.append(jnp.maximum(ms[u], jnp.where(yv >= 0.5, xv, 0.0)))
            return tuple(out)
        return body

    start(row0, 0, xb0, yb0, sx0, sy0)
    start(row0, C, xb1, yb1, sx1, sy1)

    def row_body(k, _):
        r = row0 + k
        init = tuple(jnp.full((L,), -jnp.inf, jnp.float32) for _ in range(U))
        wait(xb0, yb0, sx0, sy0)
        ms = plsc.parallel_loop(0, C, L * U, unroll=4, carry=init)(
            make_chunk(xb0, yb0))

        @pl.when(k < RPW - 1)
        def _():
            start(r + 1, 0, xb0, yb0, sx0, sy0)

        wait(xb1, yb1, sx1, sy1)
        ms = plsc.parallel_loop(0, C, L * U, unroll=4, carry=ms)(
            make_chunk(xb1, yb1))

        @pl.when(k < RPW - 1)
        def _():
            start(r + 1, C, xb1, yb1, sx1, sy1)

        m = ms[0]
        for u in range(1, U):
            m = jnp.maximum(m, ms[u])
        ob[k, :] = m
        return 0

    lax.fori_loop(0, RPW, row_body, 0)
    pltpu.sync_copy(ob, out_hbm.at[pl.ds(row0, RPW)])


def _rowmax_tc(x_ref, y_ref, o_ref):
    s = jnp.where(y_ref[...] >= 0.5, x_ref[...], 0.0)
    o_ref[...] = jnp.max(s, axis=1, keepdims=True)


_tcmax = pl.pallas_call(
    _rowmax_tc,
    grid=(B_TC // RB,),
    in_specs=[
        pl.BlockSpec((RB, N), lambda i: (i + B_SC // RB, 0)),
        pl.BlockSpec((RB, N), lambda i: (i + B_SC // RB, 0)),
    ],
    out_specs=pl.BlockSpec((RB, 1), lambda i: (i, 0)),
    out_shape=jax.ShapeDtypeStruct((B_TC, 1), jnp.float32),
)


def _combine_tc(sc_ref, tc_ref, o_ref):
    sc_sum = jnp.sum(jnp.max(sc_ref[...], axis=1))
    tc_sum = jnp.sum(tc_ref[...])
    o_ref[0, 0] = (sc_sum + tc_sum) * (1.0 / B)


_combine = pl.pallas_call(
    _combine_tc,
    out_shape=jax.ShapeDtypeStruct((1, 1), jnp.float32),
    out_specs=pl.BlockSpec(memory_space=pltpu.SMEM),
)


@jax.jit
def kernel(x, y):
    sc_partials = _hinge_sc(x, y)
    tc_max = _tcmax(x, y)
    return _combine(sc_partials, tc_max)[0, 0]
